# baseline (speedup 1.0000x reference)
import functools

import jax
import jax.numpy as jnp
from jax import lax
from jax.experimental import pallas as pl
from jax.experimental.pallas import tpu as pltpu
from jax.experimental.pallas import tpu_sc as plsc

N = 10000
DF = 128
DH = DF // 2
EPAD = 327680
CHUNK = 128
NCHROWS = EPAD // CHUNK
CPT = NCHROWS // 16
NPAD = 10240
SLAB = NPAD // 16
NBUF = 4


def _sc_prop_body(u2_hbm, rowb_hbm, col2d_hbm, zrows_hbm, out_hbm,
                  rowi_v, coli_v, gbuf_v, acc_sh,
                  sem0, sem1, sem2, sem3, ses0, ses1, ses2, ses3):
    c = lax.axis_index("c")
    s = lax.axis_index("s")
    sems = (sem0, sem1, sem2, sem3)
    ssems = (ses0, ses1, ses2, ses3)
    pltpu.sync_copy(zrows_hbm, acc_sh.at[pl.ds(s * SLAB, SLAB)])
    pltpu.sync_copy(rowb_hbm.at[c].at[pl.ds(s * CPT, CPT)], rowi_v)
    pltpu.sync_copy(col2d_hbm.at[pl.ds(s * CPT, CPT)], coli_v)
    plsc.subcore_barrier()

    def _g(jj, b):
        return pltpu.make_async_copy(u2_hbm.at[rowi_v.at[jj]],
                                     gbuf_v.at[b], sems[b])

    def _s(jj, b):
        return pltpu.make_async_copy(gbuf_v.at[b],
                                     acc_sh.at[coli_v.at[jj]], ssems[b])

    for b in range(NBUF):
        _g(b, b).start()

    def body(i, carry):
        base = i * NBUF
        for b in range(NBUF):
            _g(base + b, b).wait()
            pltpu.async_copy(gbuf_v.at[b], acc_sh.at[coli_v.at[base + b]],
                             ssems[b], add=True)
        for b in range(NBUF):
            _s(base + b, b).wait()
            _g(base + NBUF + b, b).start()
        return carry

    lax.fori_loop(0, CPT // NBUF - 1, body, 0)
    base = CPT - NBUF
    for b in range(NBUF):
        _g(base + b, b).wait()
        pltpu.async_copy(gbuf_v.at[b], acc_sh.at[coli_v.at[base + b]],
                         ssems[b], add=True)
    for b in range(NBUF):
        _s(base + b, b).wait()
    plsc.subcore_barrier()
    pltpu.sync_copy(acc_sh.at[pl.ds(s * SLAB, SLAB)],
                    out_hbm.at[c].at[pl.ds(s * SLAB, SLAB)])


_sc_prop = pl.kernel(
    _sc_prop_body,
    out_type=jax.ShapeDtypeStruct((2, NPAD, DH), jnp.float32),
    mesh=plsc.VectorSubcoreMesh(core_axis_name="c", subcore_axis_name="s"),
    compiler_params=pltpu.CompilerParams(use_tc_tiling_on_sc=False),
    scratch_types=[
        pltpu.VMEM((CPT, CHUNK), jnp.int32),
        pltpu.VMEM((CPT, CHUNK), jnp.int32),
        pltpu.VMEM((NBUF, CHUNK, DH), jnp.float32),
        pltpu.VMEM_SHARED((NPAD, DH), jnp.float32),
        pltpu.SemaphoreType.DMA,
        pltpu.SemaphoreType.DMA,
        pltpu.SemaphoreType.DMA,
        pltpu.SemaphoreType.DMA,
        pltpu.SemaphoreType.DMA,
        pltpu.SemaphoreType.DMA,
        pltpu.SemaphoreType.DMA,
        pltpu.SemaphoreType.DMA,
    ],
)


def _split_u(u_ref, v):
    u_ref[:N, :] = v[:, :DH]
    u_ref[N:, :] = v[:, DH:]


def _tc_prep_body(degp_ref, x_ref, dinv_ref, u0_ref):
    deg = degp_ref[0, :N, 0]
    dinv = jnp.where(deg > 0, lax.rsqrt(jnp.maximum(deg, 1e-12)), 0.0)
    dinv = dinv[:, None]
    dinv_ref[...] = dinv
    _split_u(u0_ref, dinv * x_ref[...])


def _tc_combine_a_body(p_ref, dinv_ref, h_ref, w0_ref, w1_ref,
                       tx_ref, u_ref, acc_ref):
    st = jnp.concatenate([p_ref[0, :N, :], p_ref[1, :N, :]], axis=1)
    dinv = dinv_ref[...]
    tx1 = -dinv * st
    tx_ref[...] = tx1
    _split_u(u_ref, dinv * tx1)
    acc_ref[...] = (jnp.dot(h_ref[...], w0_ref[...],
                            preferred_element_type=jnp.float32)
                    + jnp.dot(tx1, w1_ref[...],
                              preferred_element_type=jnp.float32))


def _tc_combine_b_body(p_ref, dinv_ref, prev2_ref, wk_ref, acc_in_ref,
                       tx_ref, u_ref, acc_ref):
    st = jnp.concatenate([p_ref[0, :N, :], p_ref[1, :N, :]], axis=1)
    dinv = dinv_ref[...]
    txk = -2.0 * dinv * st - prev2_ref[...]
    tx_ref[...] = txk
    _split_u(u_ref, dinv * txk)
    acc_ref[...] = acc_in_ref[...] + jnp.dot(
        txk, wk_ref[...], preferred_element_type=jnp.float32)


def _tc_tail_body(acc_ref, b_ref, g_ref, be_ref, dinv_ref, h_ref, u_ref):
    h = jnp.maximum(acc_ref[...] + b_ref[...][None, :], 0.0)
    m = jnp.mean(h, axis=0, keepdims=True)
    v = jnp.mean((h - m) * (h - m), axis=0, keepdims=True)
    h = (h - m) * lax.rsqrt(v + 1e-5) * g_ref[...][None, :] + be_ref[...][None, :]
    h_ref[...] = h
    _split_u(u_ref, dinv_ref[...] * h)


def _tc_final_body(acc_ref, b4_ref, mw0_ref, mg_ref, mbe_ref, mw1_ref,
                   mb1_ref, out_ref):
    h4 = acc_ref[...] + b4_ref[...][None, :]
    z = jnp.dot(h4, mw0_ref[...], preferred_element_type=jnp.float32)
    m = jnp.mean(z, axis=0, keepdims=True)
    v = jnp.mean((z - m) * (z - m), axis=0, keepdims=True)
    z = (z - m) * lax.rsqrt(v + 1e-5) * mg_ref[...][None, :] + mbe_ref[...][None, :]
    h2 = jnp.maximum(z, 0.0)
    out_ref[...] = (jnp.dot(h2, mw1_ref[...], preferred_element_type=jnp.float32)
                    + mb1_ref[...][None, :])


def _tc(body, out_shapes):
    return pl.pallas_call(body, out_shape=out_shapes)


_F = jnp.float32
_U2 = jax.ShapeDtypeStruct((2 * N, DH), _F)
_prep = _tc(_tc_prep_body, (jax.ShapeDtypeStruct((N, 1), _F), _U2))
_combine_a = _tc(_tc_combine_a_body, (jax.ShapeDtypeStruct((N, DF), _F),
                                      _U2,
                                      jax.ShapeDtypeStruct((N, DF), _F)))
_combine_b = _tc(_tc_combine_b_body, (jax.ShapeDtypeStruct((N, DF), _F),
                                      _U2,
                                      jax.ShapeDtypeStruct((N, DF), _F)))
_tail = _tc(_tc_tail_body, (jax.ShapeDtypeStruct((N, DF), _F), _U2))
_final = _tc(_tc_final_body, jax.ShapeDtypeStruct((N, 21), _F))


def kernel(x, edge_index, batch, W1, b1, W2, b2, W3, b3, W4, b4,
           g1, be1, g2, be2, g3, be3, mw0, mg, mbe, mw1, mb1):
    del batch
    pad = EPAD - edge_index.shape[1]
    rowp = jnp.concatenate(
        [edge_index[0].astype(jnp.int32), jnp.zeros((pad,), jnp.int32)])
    colp = jnp.concatenate(
        [edge_index[1].astype(jnp.int32), jnp.full((pad,), N, jnp.int32)])
    row2d = rowp.reshape(NCHROWS, CHUNK)
    rowb = jnp.stack([row2d, row2d + N])
    col2d = colp.reshape(NCHROWS, CHUNK)
    zrows = jnp.zeros((SLAB, DH), _F)
    ones2 = jnp.ones((2 * N, DH), _F)

    degp = _sc_prop(ones2, rowb, col2d, zrows)
    dinv, u = _prep(degp, x)

    h = x
    Ws = (W1, W2, W3, W4)
    bs = (b1, b2, b3, b4)
    gs = (g1, g2, g3)
    bes = (be1, be2, be3)
    for l in range(4):
        W = Ws[l]
        p = _sc_prop(u, rowb, col2d, zrows)
        tx1, u, acc = _combine_a(p, dinv, h, W[0], W[1])
        p = _sc_prop(u, rowb, col2d, zrows)
        tx2, u, acc = _combine_b(p, dinv, h, W[2], acc)
        p = _sc_prop(u, rowb, col2d, zrows)
        tx3, u, acc = _combine_b(p, dinv, tx1, W[3], acc)
        if l < 3:
            h, u = _tail(acc, bs[l], gs[l], bes[l], dinv)
        else:
            out = _final(acc, bs[l], mw0, mg, mbe, mw1, mb1)
    return out


# speedup vs baseline: 1.0764x; 1.0764x over previous
import functools

import jax
import jax.numpy as jnp
from jax import lax
from jax.experimental import pallas as pl
from jax.experimental.pallas import tpu as pltpu
from jax.experimental.pallas import tpu_sc as plsc

N = 10000
DF = 128
DH = DF // 2
EPAD = 327680
CHUNK = 128
NCHROWS = EPAD // CHUNK
EPT = EPAD // 16
NPAD = 10240
SLAB = NPAD // 16
NBUF = 4


GC = 256
GROWS = GC // CHUNK
NGRP = EPT // GC


def _sc_prop_body(u2_hbm, rowf_hbm, col3d_hbm, zrows_hbm, out_hbm,
                  rowi_v, coli_v, gbuf_v, acc_sh, sem0, sem1):
    c = lax.axis_index("c")
    s = lax.axis_index("s")
    sems = (sem0, sem1)
    pltpu.sync_copy(zrows_hbm, acc_sh.at[pl.ds(s * SLAB, SLAB)])
    pltpu.sync_copy(rowf_hbm.at[c].at[pl.ds(s * EPT, EPT)], rowi_v)
    pltpu.sync_copy(col3d_hbm.at[c].at[pl.ds(s * EPT, EPT)], coli_v)
    plsc.subcore_barrier()

    def _g(jj, b):
        return pltpu.make_async_copy(
            u2_hbm.at[rowi_v.at[pl.ds(jj * GC, GC)]], gbuf_v.at[b], sems[b])

    for b in range(2):
        _g(b, b).start()

    def body(i, carry):
        base = i * 2
        for b in range(2):
            _g(base + b, b).wait()
            pltpu.sync_copy(
                gbuf_v.at[b],
                acc_sh.at[coli_v.at[pl.ds((base + b) * GC, GC)]], add=True)
            _g(base + 2 + b, b).start()
        return carry

    lax.fori_loop(0, NGRP // 2 - 1, body, 0)
    for b in range(2):
        jj = NGRP - 2 + b
        _g(jj, b).wait()
        pltpu.sync_copy(gbuf_v.at[b],
                        acc_sh.at[coli_v.at[pl.ds(jj * GC, GC)]], add=True)
    plsc.subcore_barrier()
    pltpu.sync_copy(acc_sh.at[pl.ds(s * SLAB, SLAB)],
                    out_hbm.at[c].at[pl.ds(s * SLAB, SLAB)])


_sc_prop = pl.kernel(
    _sc_prop_body,
    out_type=jax.ShapeDtypeStruct((2, NPAD, DH), jnp.float32),
    mesh=plsc.VectorSubcoreMesh(core_axis_name="c", subcore_axis_name="s"),
    compiler_params=pltpu.CompilerParams(use_tc_tiling_on_sc=False),
    scratch_types=[
        pltpu.VMEM((EPT,), jnp.int32),
        pltpu.VMEM((EPT,), jnp.int32),
        pltpu.VMEM((2, GC, DH), jnp.float32),
        pltpu.VMEM_SHARED((NPAD, DH), jnp.float32),
        pltpu.SemaphoreType.DMA,
        pltpu.SemaphoreType.DMA,
    ],
)


def _split_u(u_ref, v):
    u_ref[:N, :] = v[:, :DH]
    u_ref[N:, :] = v[:, DH:]


def _tc_prep_body(degp_ref, x_ref, dinv_ref, u0_ref):
    deg = degp_ref[0, :N, 0]
    dinv = jnp.where(deg > 0, lax.rsqrt(jnp.maximum(deg, 1e-12)), 0.0)
    dinv = dinv[:, None]
    dinv_ref[...] = dinv
    _split_u(u0_ref, dinv * x_ref[...])


def _tc_combine_a_body(p_ref, dinv_ref, h_ref, w0_ref, w1_ref,
                       tx_ref, u_ref, acc_ref):
    st = jnp.concatenate([p_ref[0, :N, :], p_ref[1, :N, :]], axis=1)
    dinv = dinv_ref[...]
    tx1 = -dinv * st
    tx_ref[...] = tx1
    _split_u(u_ref, dinv * tx1)
    acc_ref[...] = (jnp.dot(h_ref[...], w0_ref[...],
                            preferred_element_type=jnp.float32)
                    + jnp.dot(tx1, w1_ref[...],
                              preferred_element_type=jnp.float32))


def _tc_combine_b_body(p_ref, dinv_ref, prev2_ref, wk_ref, acc_in_ref,
                       tx_ref, u_ref, acc_ref):
    st = jnp.concatenate([p_ref[0, :N, :], p_ref[1, :N, :]], axis=1)
    dinv = dinv_ref[...]
    txk = -2.0 * dinv * st - prev2_ref[...]
    tx_ref[...] = txk
    _split_u(u_ref, dinv * txk)
    acc_ref[...] = acc_in_ref[...] + jnp.dot(
        txk, wk_ref[...], preferred_element_type=jnp.float32)


def _tc_tail_body(acc_ref, b_ref, g_ref, be_ref, dinv_ref, h_ref, u_ref):
    h = jnp.maximum(acc_ref[...] + b_ref[...][None, :], 0.0)
    m = jnp.mean(h, axis=0, keepdims=True)
    v = jnp.mean((h - m) * (h - m), axis=0, keepdims=True)
    h = (h - m) * lax.rsqrt(v + 1e-5) * g_ref[...][None, :] + be_ref[...][None, :]
    h_ref[...] = h
    _split_u(u_ref, dinv_ref[...] * h)


def _tc_final_body(acc_ref, b4_ref, mw0_ref, mg_ref, mbe_ref, mw1_ref,
                   mb1_ref, out_ref):
    h4 = acc_ref[...] + b4_ref[...][None, :]
    z = jnp.dot(h4, mw0_ref[...], preferred_element_type=jnp.float32)
    m = jnp.mean(z, axis=0, keepdims=True)
    v = jnp.mean((z - m) * (z - m), axis=0, keepdims=True)
    z = (z - m) * lax.rsqrt(v + 1e-5) * mg_ref[...][None, :] + mbe_ref[...][None, :]
    h2 = jnp.maximum(z, 0.0)
    out_ref[...] = (jnp.dot(h2, mw1_ref[...], preferred_element_type=jnp.float32)
                    + mb1_ref[...][None, :])


def _tc(body, out_shapes):
    return pl.pallas_call(body, out_shape=out_shapes)


_F = jnp.float32
_U2 = jax.ShapeDtypeStruct((2 * N, DH), _F)
_prep = _tc(_tc_prep_body, (jax.ShapeDtypeStruct((N, 1), _F), _U2))
_combine_a = _tc(_tc_combine_a_body, (jax.ShapeDtypeStruct((N, DF), _F),
                                      _U2,
                                      jax.ShapeDtypeStruct((N, DF), _F)))
_combine_b = _tc(_tc_combine_b_body, (jax.ShapeDtypeStruct((N, DF), _F),
                                      _U2,
                                      jax.ShapeDtypeStruct((N, DF), _F)))
_tail = _tc(_tc_tail_body, (jax.ShapeDtypeStruct((N, DF), _F), _U2))
_final = _tc(_tc_final_body, jax.ShapeDtypeStruct((N, 21), _F))


def kernel(x, edge_index, batch, W1, b1, W2, b2, W3, b3, W4, b4,
           g1, be1, g2, be2, g3, be3, mw0, mg, mbe, mw1, mb1):
    del batch
    pad = EPAD - edge_index.shape[1]
    rowp = jnp.concatenate(
        [edge_index[0].astype(jnp.int32), jnp.zeros((pad,), jnp.int32)])
    colp = jnp.concatenate(
        [edge_index[1].astype(jnp.int32), jnp.full((pad,), N, jnp.int32)])
    rowf = jnp.stack([rowp, rowp + N])
    col3d = jnp.stack([colp, colp])
    zrows = jnp.zeros((SLAB, DH), _F)
    ones2 = jnp.ones((2 * N, DH), _F)

    degp = _sc_prop(ones2, rowf, col3d, zrows)
    dinv, u = _prep(degp, x)

    h = x
    Ws = (W1, W2, W3, W4)
    bs = (b1, b2, b3, b4)
    gs = (g1, g2, g3)
    bes = (be1, be2, be3)
    for l in range(4):
        W = Ws[l]
        p = _sc_prop(u, rowf, col3d, zrows)
        tx1, u, acc = _combine_a(p, dinv, h, W[0], W[1])
        p = _sc_prop(u, rowf, col3d, zrows)
        tx2, u, acc = _combine_b(p, dinv, h, W[2], acc)
        p = _sc_prop(u, rowf, col3d, zrows)
        tx3, u, acc = _combine_b(p, dinv, tx1, W[3], acc)
        if l < 3:
            h, u = _tail(acc, bs[l], gs[l], bes[l], dinv)
        else:
            out = _final(acc, bs[l], mw0, mg, mbe, mw1, mb1)
    return out


# speedup vs baseline: 1.6069x; 1.4928x over previous
import functools

import jax
import jax.numpy as jnp
from jax import lax
from jax.experimental import pallas as pl
from jax.experimental.pallas import tpu as pltpu
from jax.experimental.pallas import tpu_sc as plsc

N = 10000
DF = 128
DH = DF // 2
EPAD = 327680
EPT = EPAD // 16
GC = 256
NGRP = EPT // GC
PK = 2 * GC
NPAD = 10240
SLAB = NPAD // 16
TSLAB = N // 16


def _sc_prop_body(u2_hbm, pk_hbm, zrows_hbm, out_hbm,
                  ib_v, gbuf_v, table_sh, acc_sh, is0, is1, gs0, gs1):
    c = lax.axis_index("c")
    s = lax.axis_index("s")
    isems = (is0, is1)
    gsems = (gs0, gs1)
    pltpu.sync_copy(zrows_hbm, acc_sh.at[pl.ds(s * SLAB, SLAB)])
    pltpu.sync_copy(u2_hbm.at[c].at[pl.ds(s * TSLAB, TSLAB)],
                    table_sh.at[pl.ds(s * TSLAB, TSLAB)])

    def _i(j, b):
        return pltpu.make_async_copy(pk_hbm.at[s * NGRP + j], ib_v.at[b],
                                     isems[b])

    def _g(b):
        return pltpu.make_async_copy(
            table_sh.at[ib_v.at[b].at[pl.ds(0, GC)]], gbuf_v.at[b], gsems[b])

    def _scat(b):
        pltpu.sync_copy(gbuf_v.at[b], acc_sh.at[ib_v.at[b].at[pl.ds(GC, GC)]],
                        add=True)

    _i(0, 0).start()
    _i(1, 1).start()
    plsc.subcore_barrier()
    _i(0, 0).wait()
    _g(0).start()

    def body(i, carry):
        g0 = 2 * i
        _i(g0 + 1, 1).wait()
        _g(1).start()
        _g(0).wait()
        _scat(0)
        _i(g0 + 2, 0).start()
        _i(g0 + 2, 0).wait()
        _g(0).start()
        _g(1).wait()
        _scat(1)
        _i(g0 + 3, 1).start()
        return carry

    lax.fori_loop(0, NGRP // 2 - 1, body, 0)
    _i(NGRP - 1, 1).wait()
    _g(1).start()
    _g(0).wait()
    _scat(0)
    _g(1).wait()
    _scat(1)
    plsc.subcore_barrier()
    pltpu.sync_copy(acc_sh.at[pl.ds(s * SLAB, SLAB)],
                    out_hbm.at[c].at[pl.ds(s * SLAB, SLAB)])


_sc_prop = pl.kernel(
    _sc_prop_body,
    out_type=jax.ShapeDtypeStruct((2, NPAD, DH), jnp.float32),
    mesh=plsc.VectorSubcoreMesh(core_axis_name="c", subcore_axis_name="s"),
    compiler_params=pltpu.CompilerParams(use_tc_tiling_on_sc=False),
    scratch_types=[
        pltpu.VMEM((2, PK), jnp.int32),
        pltpu.VMEM((2, GC, DH), jnp.float32),
        pltpu.VMEM_SHARED((N, DH), jnp.float32),
        pltpu.VMEM_SHARED((NPAD, DH), jnp.float32),
        pltpu.SemaphoreType.DMA,
        pltpu.SemaphoreType.DMA,
        pltpu.SemaphoreType.DMA,
        pltpu.SemaphoreType.DMA,
    ],
)


def _sc_deg_body(ones_hbm, pk_hbm, zrows_hbm, out_hbm,
                 ib_v, obuf_v, acc_sh, is0, is1):
    c = lax.axis_index("c")
    s = lax.axis_index("s")
    isems = (is0, is1)
    pltpu.sync_copy(zrows_hbm, acc_sh.at[pl.ds(s * SLAB, SLAB)])
    pltpu.sync_copy(ones_hbm, obuf_v)

    def _i(j, b):
        return pltpu.make_async_copy(pk_hbm.at[s * NGRP + j], ib_v.at[b],
                                     isems[b])

    _i(0, 0).start()
    _i(1, 1).start()
    plsc.subcore_barrier()

    def body(i, carry):
        g0 = 2 * i
        _i(g0, 0).wait()
        pltpu.sync_copy(obuf_v, acc_sh.at[ib_v.at[0].at[pl.ds(GC, GC)]],
                        add=True)
        _i(g0 + 2, 0).start()
        _i(g0 + 1, 1).wait()
        pltpu.sync_copy(obuf_v, acc_sh.at[ib_v.at[1].at[pl.ds(GC, GC)]],
                        add=True)
        _i(g0 + 3, 1).start()
        return carry

    lax.fori_loop(0, NGRP // 2 - 1, body, 0)
    for b in range(2):
        _i(NGRP - 2 + b, b).wait()
        pltpu.sync_copy(obuf_v, acc_sh.at[ib_v.at[b].at[pl.ds(GC, GC)]],
                        add=True)
    plsc.subcore_barrier()
    pltpu.sync_copy(acc_sh.at[pl.ds(s * SLAB, SLAB)],
                    out_hbm.at[c].at[pl.ds(s * SLAB, SLAB)])


_sc_deg = pl.kernel(
    _sc_deg_body,
    out_type=jax.ShapeDtypeStruct((2, NPAD, DH), jnp.float32),
    mesh=plsc.VectorSubcoreMesh(core_axis_name="c", subcore_axis_name="s"),
    compiler_params=pltpu.CompilerParams(use_tc_tiling_on_sc=False),
    scratch_types=[
        pltpu.VMEM((2, PK), jnp.int32),
        pltpu.VMEM((GC, DH), jnp.float32),
        pltpu.VMEM_SHARED((NPAD, DH), jnp.float32),
        pltpu.SemaphoreType.DMA,
        pltpu.SemaphoreType.DMA,
    ],
)


def _split_u(u_ref, v):
    u_ref[0] = v[:, :DH]
    u_ref[1] = v[:, DH:]


def _tc_prep_body(degp_ref, x_ref, dinv_ref, u0_ref):
    deg = degp_ref[0, :N, 0]
    dinv = jnp.where(deg > 0, lax.rsqrt(jnp.maximum(deg, 1e-12)), 0.0)
    dinv = dinv[:, None]
    dinv_ref[...] = dinv
    _split_u(u0_ref, dinv * x_ref[...])


def _tc_combine_a_body(p_ref, dinv_ref, h_ref, w0_ref, w1_ref,
                       tx_ref, u_ref, acc_ref):
    st = jnp.concatenate([p_ref[0, :N, :], p_ref[1, :N, :]], axis=1)
    dinv = dinv_ref[...]
    tx1 = -dinv * st
    tx_ref[...] = tx1
    _split_u(u_ref, dinv * tx1)
    acc_ref[...] = (jnp.dot(h_ref[...], w0_ref[...],
                            preferred_element_type=jnp.float32)
                    + jnp.dot(tx1, w1_ref[...],
                              preferred_element_type=jnp.float32))


def _tc_combine_b_body(p_ref, dinv_ref, prev2_ref, wk_ref, acc_in_ref,
                       tx_ref, u_ref, acc_ref):
    st = jnp.concatenate([p_ref[0, :N, :], p_ref[1, :N, :]], axis=1)
    dinv = dinv_ref[...]
    txk = -2.0 * dinv * st - prev2_ref[...]
    tx_ref[...] = txk
    _split_u(u_ref, dinv * txk)
    acc_ref[...] = acc_in_ref[...] + jnp.dot(
        txk, wk_ref[...], preferred_element_type=jnp.float32)


def _tc_tail_body(acc_ref, b_ref, g_ref, be_ref, dinv_ref, h_ref, u_ref):
    h = jnp.maximum(acc_ref[...] + b_ref[...][None, :], 0.0)
    m = jnp.mean(h, axis=0, keepdims=True)
    v = jnp.mean((h - m) * (h - m), axis=0, keepdims=True)
    h = (h - m) * lax.rsqrt(v + 1e-5) * g_ref[...][None, :] + be_ref[...][None, :]
    h_ref[...] = h
    _split_u(u_ref, dinv_ref[...] * h)


def _tc_final_body(acc_ref, b4_ref, mw0_ref, mg_ref, mbe_ref, mw1_ref,
                   mb1_ref, out_ref):
    h4 = acc_ref[...] + b4_ref[...][None, :]
    z = jnp.dot(h4, mw0_ref[...], preferred_element_type=jnp.float32)
    m = jnp.mean(z, axis=0, keepdims=True)
    v = jnp.mean((z - m) * (z - m), axis=0, keepdims=True)
    z = (z - m) * lax.rsqrt(v + 1e-5) * mg_ref[...][None, :] + mbe_ref[...][None, :]
    h2 = jnp.maximum(z, 0.0)
    out_ref[...] = (jnp.dot(h2, mw1_ref[...], preferred_element_type=jnp.float32)
                    + mb1_ref[...][None, :])


def _tc(body, out_shapes):
    return pl.pallas_call(body, out_shape=out_shapes)


_F = jnp.float32
_U2 = jax.ShapeDtypeStruct((2, N, DH), _F)
_prep = _tc(_tc_prep_body, (jax.ShapeDtypeStruct((N, 1), _F), _U2))
_combine_a = _tc(_tc_combine_a_body, (jax.ShapeDtypeStruct((N, DF), _F),
                                      _U2,
                                      jax.ShapeDtypeStruct((N, DF), _F)))
_combine_b = _tc(_tc_combine_b_body, (jax.ShapeDtypeStruct((N, DF), _F),
                                      _U2,
                                      jax.ShapeDtypeStruct((N, DF), _F)))
_tail = _tc(_tc_tail_body, (jax.ShapeDtypeStruct((N, DF), _F), _U2))
_final = _tc(_tc_final_body, jax.ShapeDtypeStruct((N, 21), _F))


def kernel(x, edge_index, batch, W1, b1, W2, b2, W3, b3, W4, b4,
           g1, be1, g2, be2, g3, be3, mw0, mg, mbe, mw1, mb1):
    del batch
    pad = EPAD - edge_index.shape[1]
    rowp = jnp.concatenate(
        [edge_index[0].astype(jnp.int32), jnp.zeros((pad,), jnp.int32)])
    colp = jnp.concatenate(
        [edge_index[1].astype(jnp.int32), jnp.full((pad,), N, jnp.int32)])
    pk = jnp.concatenate([rowp.reshape(16 * NGRP, 1, GC),
                          colp.reshape(16 * NGRP, 1, GC)],
                         axis=1).reshape(16 * NGRP, PK)
    zrows = jnp.zeros((SLAB, DH), _F)
    onesg = jnp.ones((GC, DH), _F)

    degp = _sc_deg(onesg, pk, zrows)
    dinv, u = _prep(degp, x)

    h = x
    Ws = (W1, W2, W3, W4)
    bs = (b1, b2, b3, b4)
    gs = (g1, g2, g3)
    bes = (be1, be2, be3)
    for l in range(4):
        W = Ws[l]
        p = _sc_prop(u, pk, zrows)
        tx1, u, acc = _combine_a(p, dinv, h, W[0], W[1])
        p = _sc_prop(u, pk, zrows)
        tx2, u, acc = _combine_b(p, dinv, h, W[2], acc)
        p = _sc_prop(u, pk, zrows)
        tx3, u, acc = _combine_b(p, dinv, tx1, W[3], acc)
        if l < 3:
            h, u = _tail(acc, bs[l], gs[l], bes[l], dinv)
        else:
            out = _final(acc, bs[l], mw0, mg, mbe, mw1, mb1)
    return out


# speedup vs baseline: 1.6380x; 1.0194x over previous
import functools

import jax
import jax.numpy as jnp
from jax import lax
from jax.experimental import pallas as pl
from jax.experimental.pallas import tpu as pltpu
from jax.experimental.pallas import tpu_sc as plsc

N = 10000
DF = 128
DH = DF // 2
EPAD = 327680
EPT = EPAD // 16
GC = 256
NGRP = EPT // GC
PK = 2 * GC
NPAD = 10240
SLAB = NPAD // 16
TSLAB = N // 16


def _sc_prop_body(u2_hbm, pk_hbm, zrows_hbm, out_hbm,
                  ib_v, gbuf_v, table_sh, acc_sh, is0, is1, gs0, gs1):
    c = lax.axis_index("c")
    s = lax.axis_index("s")
    isems = (is0, is1)
    gsems = (gs0, gs1)
    pltpu.sync_copy(zrows_hbm, acc_sh.at[pl.ds(s * SLAB, SLAB)])
    pltpu.sync_copy(u2_hbm.at[c].at[pl.ds(s * TSLAB, TSLAB)],
                    table_sh.at[pl.ds(s * TSLAB, TSLAB)])

    def _i(j, b):
        return pltpu.make_async_copy(pk_hbm.at[s * NGRP + j], ib_v.at[b],
                                     isems[b])

    def _g(b):
        return pltpu.make_async_copy(
            table_sh.at[ib_v.at[b].at[pl.ds(0, GC)]], gbuf_v.at[b], gsems[b])

    def _scat(b):
        pltpu.sync_copy(gbuf_v.at[b], acc_sh.at[ib_v.at[b].at[pl.ds(GC, GC)]],
                        add=True)

    _i(0, 0).start()
    _i(1, 1).start()
    plsc.subcore_barrier()
    _i(0, 0).wait()
    _g(0).start()

    def body(i, carry):
        g0 = 2 * i
        _i(g0 + 1, 1).wait()
        _g(1).start()
        _g(0).wait()
        _scat(0)
        _i(g0 + 2, 0).start()
        _i(g0 + 2, 0).wait()
        _g(0).start()
        _g(1).wait()
        _scat(1)
        _i(g0 + 3, 1).start()
        return carry

    lax.fori_loop(0, NGRP // 2 - 1, body, 0)
    _i(NGRP - 1, 1).wait()
    _g(1).start()
    _g(0).wait()
    _scat(0)
    _g(1).wait()
    _scat(1)
    plsc.subcore_barrier()
    pltpu.sync_copy(acc_sh.at[pl.ds(s * SLAB, SLAB)],
                    out_hbm.at[c].at[pl.ds(s * SLAB, SLAB)])


_sc_prop = pl.kernel(
    _sc_prop_body,
    out_type=jax.ShapeDtypeStruct((2, NPAD, DH), jnp.float32),
    mesh=plsc.VectorSubcoreMesh(core_axis_name="c", subcore_axis_name="s"),
    compiler_params=pltpu.CompilerParams(use_tc_tiling_on_sc=False),
    scratch_types=[
        pltpu.VMEM((2, PK), jnp.int32),
        pltpu.VMEM((2, GC, DH), jnp.float32),
        pltpu.VMEM_SHARED((N, DH), jnp.float32),
        pltpu.VMEM_SHARED((NPAD, DH), jnp.float32),
        pltpu.SemaphoreType.DMA,
        pltpu.SemaphoreType.DMA,
        pltpu.SemaphoreType.DMA,
        pltpu.SemaphoreType.DMA,
    ],
)


def _sc_deg_body(ones_hbm, pk_hbm, zrows_hbm, out_hbm,
                 ib_v, obuf_v, acc_sh, is0, is1):
    c = lax.axis_index("c")
    s = lax.axis_index("s")
    isems = (is0, is1)
    pltpu.sync_copy(zrows_hbm, acc_sh.at[pl.ds(s * SLAB, SLAB)])
    pltpu.sync_copy(ones_hbm, obuf_v)

    def _i(j, b):
        return pltpu.make_async_copy(pk_hbm.at[s * NGRP + j], ib_v.at[b],
                                     isems[b])

    _i(0, 0).start()
    _i(1, 1).start()
    plsc.subcore_barrier()

    def body(i, carry):
        g0 = 2 * i
        _i(g0, 0).wait()
        pltpu.sync_copy(obuf_v, acc_sh.at[ib_v.at[0].at[pl.ds(GC, GC)]],
                        add=True)
        _i(g0 + 2, 0).start()
        _i(g0 + 1, 1).wait()
        pltpu.sync_copy(obuf_v, acc_sh.at[ib_v.at[1].at[pl.ds(GC, GC)]],
                        add=True)
        _i(g0 + 3, 1).start()
        return carry

    lax.fori_loop(0, NGRP // 2 - 1, body, 0)
    for b in range(2):
        _i(NGRP - 2 + b, b).wait()
        pltpu.sync_copy(obuf_v, acc_sh.at[ib_v.at[b].at[pl.ds(GC, GC)]],
                        add=True)
    plsc.subcore_barrier()
    pltpu.sync_copy(acc_sh.at[pl.ds(s * SLAB, SLAB)],
                    out_hbm.at[c].at[pl.ds(s * SLAB, SLAB)])


_sc_deg = pl.kernel(
    _sc_deg_body,
    out_type=jax.ShapeDtypeStruct((2, NPAD, DH), jnp.float32),
    mesh=plsc.VectorSubcoreMesh(core_axis_name="c", subcore_axis_name="s"),
    compiler_params=pltpu.CompilerParams(use_tc_tiling_on_sc=False),
    scratch_types=[
        pltpu.VMEM((2, PK), jnp.int32),
        pltpu.VMEM((GC, DH), jnp.float32),
        pltpu.VMEM_SHARED((NPAD, DH), jnp.float32),
        pltpu.SemaphoreType.DMA,
        pltpu.SemaphoreType.DMA,
    ],
)


def _split_u(u_ref, v):
    u_ref[0] = v[:, :DH]
    u_ref[1] = v[:, DH:]


def _tc_prep_body(degp_ref, x_ref, dinv_ref, u0_ref):
    deg = degp_ref[0, :N, 0]
    dinv = jnp.where(deg > 0, lax.rsqrt(jnp.maximum(deg, 1e-12)), 0.0)
    dinv = dinv[:, None]
    dinv_ref[...] = dinv
    _split_u(u0_ref, dinv * x_ref[...])


def _tc_combine_a_body(p_ref, dinv_ref, h_ref, w0_ref, w1_ref,
                       tx_ref, u_ref, acc_ref):
    st = jnp.concatenate([p_ref[0, :N, :], p_ref[1, :N, :]], axis=1)
    dinv = dinv_ref[...]
    tx1 = -dinv * st
    tx_ref[...] = tx1
    _split_u(u_ref, dinv * tx1)
    acc_ref[...] = (jnp.dot(h_ref[...], w0_ref[...],
                            preferred_element_type=jnp.float32)
                    + jnp.dot(tx1, w1_ref[...],
                              preferred_element_type=jnp.float32))


def _tc_combine_b_body(p_ref, dinv_ref, prev2_ref, wk_ref, acc_in_ref,
                       u_ref, acc_ref):
    st = jnp.concatenate([p_ref[0, :N, :], p_ref[1, :N, :]], axis=1)
    dinv = dinv_ref[...]
    txk = -2.0 * dinv * st - prev2_ref[...]
    _split_u(u_ref, dinv * txk)
    acc_ref[...] = acc_in_ref[...] + jnp.dot(
        txk, wk_ref[...], preferred_element_type=jnp.float32)


def _tc_combine_b_tail_body(p_ref, dinv_ref, prev2_ref, wk_ref, acc_in_ref,
                            b_ref, g_ref, be_ref, h_ref, u_ref):
    st = jnp.concatenate([p_ref[0, :N, :], p_ref[1, :N, :]], axis=1)
    dinv = dinv_ref[...]
    txk = -2.0 * dinv * st - prev2_ref[...]
    acc = acc_in_ref[...] + jnp.dot(
        txk, wk_ref[...], preferred_element_type=jnp.float32)
    h = jnp.maximum(acc + b_ref[...][None, :], 0.0)
    m = jnp.mean(h, axis=0, keepdims=True)
    v = jnp.mean((h - m) * (h - m), axis=0, keepdims=True)
    h = (h - m) * lax.rsqrt(v + 1e-5) * g_ref[...][None, :] + be_ref[...][None, :]
    h_ref[...] = h
    _split_u(u_ref, dinv * h)


def _tc_combine_b_final_body(p_ref, dinv_ref, prev2_ref, wk_ref, acc_in_ref,
                             b4_ref, mw0_ref, mg_ref, mbe_ref, mw1_ref,
                             mb1_ref, out_ref):
    st = jnp.concatenate([p_ref[0, :N, :], p_ref[1, :N, :]], axis=1)
    dinv = dinv_ref[...]
    txk = -2.0 * dinv * st - prev2_ref[...]
    acc = acc_in_ref[...] + jnp.dot(
        txk, wk_ref[...], preferred_element_type=jnp.float32)
    h4 = acc + b4_ref[...][None, :]
    z = jnp.dot(h4, mw0_ref[...], preferred_element_type=jnp.float32)
    m = jnp.mean(z, axis=0, keepdims=True)
    v = jnp.mean((z - m) * (z - m), axis=0, keepdims=True)
    z = (z - m) * lax.rsqrt(v + 1e-5) * mg_ref[...][None, :] + mbe_ref[...][None, :]
    h2 = jnp.maximum(z, 0.0)
    out_ref[...] = (jnp.dot(h2, mw1_ref[...], preferred_element_type=jnp.float32)
                    + mb1_ref[...][None, :])


def _tc(body, out_shapes):
    return pl.pallas_call(body, out_shape=out_shapes)


_F = jnp.float32
_U2 = jax.ShapeDtypeStruct((2, N, DH), _F)
_prep = _tc(_tc_prep_body, (jax.ShapeDtypeStruct((N, 1), _F), _U2))
_combine_a = _tc(_tc_combine_a_body, (jax.ShapeDtypeStruct((N, DF), _F),
                                      _U2,
                                      jax.ShapeDtypeStruct((N, DF), _F)))
_combine_b = _tc(_tc_combine_b_body, (_U2,
                                      jax.ShapeDtypeStruct((N, DF), _F)))
_combine_bt = _tc(_tc_combine_b_tail_body,
                  (jax.ShapeDtypeStruct((N, DF), _F), _U2))
_combine_bf = _tc(_tc_combine_b_final_body, jax.ShapeDtypeStruct((N, 21), _F))


def kernel(x, edge_index, batch, W1, b1, W2, b2, W3, b3, W4, b4,
           g1, be1, g2, be2, g3, be3, mw0, mg, mbe, mw1, mb1):
    del batch
    pad = EPAD - edge_index.shape[1]
    rowp = jnp.concatenate(
        [edge_index[0].astype(jnp.int32), jnp.zeros((pad,), jnp.int32)])
    colp = jnp.concatenate(
        [edge_index[1].astype(jnp.int32), jnp.full((pad,), N, jnp.int32)])
    pk = jnp.concatenate([rowp.reshape(16 * NGRP, 1, GC),
                          colp.reshape(16 * NGRP, 1, GC)],
                         axis=1).reshape(16 * NGRP, PK)
    zrows = jnp.zeros((SLAB, DH), _F)
    onesg = jnp.ones((GC, DH), _F)

    degp = _sc_deg(onesg, pk, zrows)
    dinv, u = _prep(degp, x)

    h = x
    Ws = (W1, W2, W3, W4)
    bs = (b1, b2, b3, b4)
    gs = (g1, g2, g3)
    bes = (be1, be2, be3)
    for l in range(4):
        W = Ws[l]
        p = _sc_prop(u, pk, zrows)
        tx1, u, acc = _combine_a(p, dinv, h, W[0], W[1])
        p = _sc_prop(u, pk, zrows)
        u, acc = _combine_b(p, dinv, h, W[2], acc)
        p = _sc_prop(u, pk, zrows)
        if l < 3:
            h, u = _combine_bt(p, dinv, tx1, W[3], acc,
                               bs[l], gs[l], bes[l])
        else:
            out = _combine_bf(p, dinv, tx1, W[3], acc,
                              bs[l], mw0, mg, mbe, mw1, mb1)
    return out


# speedup vs baseline: 1.6412x; 1.0020x over previous
import functools

import jax
import jax.numpy as jnp
from jax import lax
from jax.experimental import pallas as pl
from jax.experimental.pallas import tpu as pltpu
from jax.experimental.pallas import tpu_sc as plsc

N = 10000
DF = 128
DH = DF // 2
EPAD = 327680
EPT = EPAD // 16
GC = 256
NGRP = EPT // GC
PK = 2 * GC
NPAD = 10240
SLAB = NPAD // 16
TSLAB = N // 16


def _sc_prop_body(u2_hbm, pk_hbm, zrows_hbm, out_hbm,
                  ib_v, gbuf_v, table_sh, acc_sh, is0, is1, gs0, gs1):
    c = lax.axis_index("c")
    s = lax.axis_index("s")
    isems = (is0, is1)
    gsems = (gs0, gs1)
    pltpu.sync_copy(zrows_hbm, acc_sh.at[pl.ds(s * SLAB, SLAB)])
    pltpu.sync_copy(u2_hbm.at[c].at[pl.ds(s * TSLAB, TSLAB)],
                    table_sh.at[pl.ds(s * TSLAB, TSLAB)])

    def _i(j, b):
        return pltpu.make_async_copy(pk_hbm.at[s * NGRP + j], ib_v.at[b],
                                     isems[b])

    def _g(b):
        return pltpu.make_async_copy(
            table_sh.at[ib_v.at[b].at[pl.ds(0, GC)]], gbuf_v.at[b], gsems[b])

    def _scat(b):
        pltpu.sync_copy(gbuf_v.at[b], acc_sh.at[ib_v.at[b].at[pl.ds(GC, GC)]],
                        add=True)

    _i(0, 0).start()
    _i(1, 1).start()
    plsc.subcore_barrier()
    _i(0, 0).wait()
    _g(0).start()

    def body(i, carry):
        g0 = 2 * i
        _i(g0 + 1, 1).wait()
        _g(1).start()
        _g(0).wait()
        _scat(0)
        _i(g0 + 2, 0).start()
        _i(g0 + 2, 0).wait()
        _g(0).start()
        _g(1).wait()
        _scat(1)
        _i(g0 + 3, 1).start()
        return carry

    lax.fori_loop(0, NGRP // 2 - 1, body, 0)
    _i(NGRP - 1, 1).wait()
    _g(1).start()
    _g(0).wait()
    _scat(0)
    _g(1).wait()
    _scat(1)
    plsc.subcore_barrier()
    pltpu.sync_copy(acc_sh.at[pl.ds(s * SLAB, SLAB)],
                    out_hbm.at[c].at[pl.ds(s * SLAB, SLAB)])


_sc_prop = pl.kernel(
    _sc_prop_body,
    out_type=jax.ShapeDtypeStruct((2, NPAD, DH), jnp.float32),
    mesh=plsc.VectorSubcoreMesh(core_axis_name="c", subcore_axis_name="s"),
    compiler_params=pltpu.CompilerParams(use_tc_tiling_on_sc=False),
    scratch_types=[
        pltpu.VMEM((2, PK), jnp.int32),
        pltpu.VMEM((2, GC, DH), jnp.float32),
        pltpu.VMEM_SHARED((N, DH), jnp.float32),
        pltpu.VMEM_SHARED((NPAD, DH), jnp.float32),
        pltpu.SemaphoreType.DMA,
        pltpu.SemaphoreType.DMA,
        pltpu.SemaphoreType.DMA,
        pltpu.SemaphoreType.DMA,
    ],
)


def _sc_deg_body(ones_hbm, pk_hbm, zrows_hbm, out_hbm,
                 ib_v, obuf_v, acc_sh, is0, is1):
    c = lax.axis_index("c")
    s = lax.axis_index("s")
    isems = (is0, is1)
    pltpu.sync_copy(zrows_hbm, acc_sh.at[pl.ds(s * SLAB, SLAB)])
    pltpu.sync_copy(ones_hbm, obuf_v)

    def _i(j, b):
        return pltpu.make_async_copy(pk_hbm.at[s * NGRP + j], ib_v.at[b],
                                     isems[b])

    _i(0, 0).start()
    _i(1, 1).start()
    plsc.subcore_barrier()

    def body(i, carry):
        g0 = 2 * i
        _i(g0, 0).wait()
        pltpu.sync_copy(obuf_v, acc_sh.at[ib_v.at[0].at[pl.ds(GC, GC)]],
                        add=True)
        _i(g0 + 2, 0).start()
        _i(g0 + 1, 1).wait()
        pltpu.sync_copy(obuf_v, acc_sh.at[ib_v.at[1].at[pl.ds(GC, GC)]],
                        add=True)
        _i(g0 + 3, 1).start()
        return carry

    lax.fori_loop(0, NGRP // 2 - 1, body, 0)
    for b in range(2):
        _i(NGRP - 2 + b, b).wait()
        pltpu.sync_copy(obuf_v, acc_sh.at[ib_v.at[b].at[pl.ds(GC, GC)]],
                        add=True)
    plsc.subcore_barrier()
    pltpu.sync_copy(acc_sh.at[pl.ds(s * SLAB, SLAB)],
                    out_hbm.at[c].at[pl.ds(s * SLAB, SLAB)])


_sc_deg = pl.kernel(
    _sc_deg_body,
    out_type=jax.ShapeDtypeStruct((2, NPAD, DH), jnp.float32),
    mesh=plsc.VectorSubcoreMesh(core_axis_name="c", subcore_axis_name="s"),
    compiler_params=pltpu.CompilerParams(use_tc_tiling_on_sc=False),
    scratch_types=[
        pltpu.VMEM((2, PK), jnp.int32),
        pltpu.VMEM((GC, DH), jnp.float32),
        pltpu.VMEM_SHARED((NPAD, DH), jnp.float32),
        pltpu.SemaphoreType.DMA,
        pltpu.SemaphoreType.DMA,
    ],
)


def _split_u(u_ref, v):
    u_ref[0] = v[:, :DH]
    u_ref[1] = v[:, DH:]


def _tc_prep_body(degp_ref, x_ref, dinv_ref, u0_ref):
    deg = degp_ref[0, :N, 0]
    dinv = jnp.where(deg > 0, lax.rsqrt(jnp.maximum(deg, 1e-12)), 0.0)
    dinv = dinv[:, None]
    dinv_ref[...] = dinv
    _split_u(u0_ref, dinv * x_ref[...])


def _tc_combine_a_u_body(p_ref, dinv_ref, tx_ref, u_ref):
    st = jnp.concatenate([p_ref[0, :N, :], p_ref[1, :N, :]], axis=1)
    dinv = dinv_ref[...]
    tx1 = -dinv * st
    tx_ref[...] = tx1
    _split_u(u_ref, dinv * tx1)


def _tc_combine_a_mm_body(h_ref, tx_ref, w0_ref, w1_ref, acc_ref):
    acc_ref[...] = (jnp.dot(h_ref[...], w0_ref[...],
                            preferred_element_type=jnp.float32)
                    + jnp.dot(tx_ref[...], w1_ref[...],
                              preferred_element_type=jnp.float32))


def _tc_combine_b_u_body(p_ref, dinv_ref, prev2_ref, tx_ref, u_ref):
    st = jnp.concatenate([p_ref[0, :N, :], p_ref[1, :N, :]], axis=1)
    dinv = dinv_ref[...]
    txk = -2.0 * dinv * st - prev2_ref[...]
    tx_ref[...] = txk
    _split_u(u_ref, dinv * txk)


def _tc_combine_b_mm_body(acc_in_ref, tx_ref, wk_ref, acc_ref):
    acc_ref[...] = acc_in_ref[...] + jnp.dot(
        tx_ref[...], wk_ref[...], preferred_element_type=jnp.float32)


def _tc_combine_b_tail_body(p_ref, dinv_ref, prev2_ref, wk_ref, acc_in_ref,
                            b_ref, g_ref, be_ref, h_ref, u_ref):
    st = jnp.concatenate([p_ref[0, :N, :], p_ref[1, :N, :]], axis=1)
    dinv = dinv_ref[...]
    txk = -2.0 * dinv * st - prev2_ref[...]
    acc = acc_in_ref[...] + jnp.dot(
        txk, wk_ref[...], preferred_element_type=jnp.float32)
    h = jnp.maximum(acc + b_ref[...][None, :], 0.0)
    m = jnp.mean(h, axis=0, keepdims=True)
    v = jnp.mean((h - m) * (h - m), axis=0, keepdims=True)
    h = (h - m) * lax.rsqrt(v + 1e-5) * g_ref[...][None, :] + be_ref[...][None, :]
    h_ref[...] = h
    _split_u(u_ref, dinv * h)


def _tc_combine_b_final_body(p_ref, dinv_ref, prev2_ref, wk_ref, acc_in_ref,
                             b4_ref, mw0_ref, mg_ref, mbe_ref, mw1_ref,
                             mb1_ref, out_ref):
    st = jnp.concatenate([p_ref[0, :N, :], p_ref[1, :N, :]], axis=1)
    dinv = dinv_ref[...]
    txk = -2.0 * dinv * st - prev2_ref[...]
    acc = acc_in_ref[...] + jnp.dot(
        txk, wk_ref[...], preferred_element_type=jnp.float32)
    h4 = acc + b4_ref[...][None, :]
    z = jnp.dot(h4, mw0_ref[...], preferred_element_type=jnp.float32)
    m = jnp.mean(z, axis=0, keepdims=True)
    v = jnp.mean((z - m) * (z - m), axis=0, keepdims=True)
    z = (z - m) * lax.rsqrt(v + 1e-5) * mg_ref[...][None, :] + mbe_ref[...][None, :]
    h2 = jnp.maximum(z, 0.0)
    out_ref[...] = (jnp.dot(h2, mw1_ref[...], preferred_element_type=jnp.float32)
                    + mb1_ref[...][None, :])


def _tc(body, out_shapes):
    return pl.pallas_call(body, out_shape=out_shapes)


_F = jnp.float32
_U2 = jax.ShapeDtypeStruct((2, N, DH), _F)
_prep = _tc(_tc_prep_body, (jax.ShapeDtypeStruct((N, 1), _F), _U2))
_NF = jax.ShapeDtypeStruct((N, DF), _F)
_combine_a_u = _tc(_tc_combine_a_u_body, (_NF, _U2))
_combine_a_mm = _tc(_tc_combine_a_mm_body, _NF)
_combine_b_u = _tc(_tc_combine_b_u_body, (_NF, _U2))
_combine_b_mm = _tc(_tc_combine_b_mm_body, _NF)
_combine_bt = _tc(_tc_combine_b_tail_body,
                  (jax.ShapeDtypeStruct((N, DF), _F), _U2))
_combine_bf = _tc(_tc_combine_b_final_body, jax.ShapeDtypeStruct((N, 21), _F))


def kernel(x, edge_index, batch, W1, b1, W2, b2, W3, b3, W4, b4,
           g1, be1, g2, be2, g3, be3, mw0, mg, mbe, mw1, mb1):
    del batch
    pad = EPAD - edge_index.shape[1]
    rowp = jnp.concatenate(
        [edge_index[0].astype(jnp.int32), jnp.zeros((pad,), jnp.int32)])
    colp = jnp.concatenate(
        [edge_index[1].astype(jnp.int32), jnp.full((pad,), N, jnp.int32)])
    pk = jnp.concatenate([rowp.reshape(16 * NGRP, 1, GC),
                          colp.reshape(16 * NGRP, 1, GC)],
                         axis=1).reshape(16 * NGRP, PK)
    zrows = jnp.zeros((SLAB, DH), _F)
    onesg = jnp.ones((GC, DH), _F)

    degp = _sc_deg(onesg, pk, zrows)
    dinv, u = _prep(degp, x)

    h = x
    Ws = (W1, W2, W3, W4)
    bs = (b1, b2, b3, b4)
    gs = (g1, g2, g3)
    bes = (be1, be2, be3)
    for l in range(4):
        W = Ws[l]
        p = _sc_prop(u, pk, zrows)
        tx1, u = _combine_a_u(p, dinv)
        acc = _combine_a_mm(h, tx1, W[0], W[1])
        p = _sc_prop(u, pk, zrows)
        tx2, u = _combine_b_u(p, dinv, h)
        acc = _combine_b_mm(acc, tx2, W[2])
        p = _sc_prop(u, pk, zrows)
        if l < 3:
            h, u = _combine_bt(p, dinv, tx1, W[3], acc,
                               bs[l], gs[l], bes[l])
        else:
            out = _combine_bf(p, dinv, tx1, W[3], acc,
                              bs[l], mw0, mg, mbe, mw1, mb1)
    return out


# speedup vs baseline: 1.6787x; 1.0228x over previous
import functools

import jax
import jax.numpy as jnp
from jax import lax
from jax.experimental import pallas as pl
from jax.experimental.pallas import tpu as pltpu
from jax.experimental.pallas import tpu_sc as plsc

N = 10000
DF = 128
DH = DF // 2
EPAD = 327680
EPT = EPAD // 16
GC = 320
NGRP = EPT // GC
PK = 2 * GC
NPAD = 10240
SLAB = NPAD // 16
TSLAB = N // 16


def _sc_prop_body(u2_hbm, pk_hbm, zrows_hbm, out_hbm,
                  ib_v, gbuf_v, table_sh, acc_sh, is0, is1, gs0, gs1):
    c = lax.axis_index("c")
    s = lax.axis_index("s")
    isems = (is0, is1)
    gsems = (gs0, gs1)
    pltpu.sync_copy(zrows_hbm, acc_sh.at[pl.ds(s * SLAB, SLAB)])
    pltpu.sync_copy(u2_hbm.at[c].at[pl.ds(s * TSLAB, TSLAB)],
                    table_sh.at[pl.ds(s * TSLAB, TSLAB)])

    def _i(j, b):
        return pltpu.make_async_copy(pk_hbm.at[s * NGRP + j], ib_v.at[b],
                                     isems[b])

    def _g(b):
        return pltpu.make_async_copy(
            table_sh.at[ib_v.at[b].at[pl.ds(0, GC)]], gbuf_v.at[b], gsems[b])

    def _scat(b):
        pltpu.sync_copy(gbuf_v.at[b], acc_sh.at[ib_v.at[b].at[pl.ds(GC, GC)]],
                        add=True)

    _i(0, 0).start()
    _i(1, 1).start()
    plsc.subcore_barrier()
    _i(0, 0).wait()
    _g(0).start()

    def body(i, carry):
        g0 = 2 * i
        _i(g0 + 1, 1).wait()
        _g(1).start()
        _g(0).wait()
        _scat(0)
        _i(g0 + 2, 0).start()
        _i(g0 + 2, 0).wait()
        _g(0).start()
        _g(1).wait()
        _scat(1)
        _i(g0 + 3, 1).start()
        return carry

    lax.fori_loop(0, NGRP // 2 - 1, body, 0)
    _i(NGRP - 1, 1).wait()
    _g(1).start()
    _g(0).wait()
    _scat(0)
    _g(1).wait()
    _scat(1)
    plsc.subcore_barrier()
    pltpu.sync_copy(acc_sh.at[pl.ds(s * SLAB, SLAB)],
                    out_hbm.at[c].at[pl.ds(s * SLAB, SLAB)])


_sc_prop = pl.kernel(
    _sc_prop_body,
    out_type=jax.ShapeDtypeStruct((2, NPAD, DH), jnp.float32),
    mesh=plsc.VectorSubcoreMesh(core_axis_name="c", subcore_axis_name="s"),
    compiler_params=pltpu.CompilerParams(use_tc_tiling_on_sc=False),
    scratch_types=[
        pltpu.VMEM((2, PK), jnp.int32),
        pltpu.VMEM((2, GC, DH), jnp.float32),
        pltpu.VMEM_SHARED((N, DH), jnp.float32),
        pltpu.VMEM_SHARED((NPAD, DH), jnp.float32),
        pltpu.SemaphoreType.DMA,
        pltpu.SemaphoreType.DMA,
        pltpu.SemaphoreType.DMA,
        pltpu.SemaphoreType.DMA,
    ],
)


def _sc_deg_body(ones_hbm, pk_hbm, zrows_hbm, out_hbm,
                 ib_v, obuf_v, acc_sh, is0, is1):
    c = lax.axis_index("c")
    s = lax.axis_index("s")
    isems = (is0, is1)
    pltpu.sync_copy(zrows_hbm, acc_sh.at[pl.ds(s * SLAB, SLAB)])
    pltpu.sync_copy(ones_hbm, obuf_v)

    def _i(j, b):
        return pltpu.make_async_copy(pk_hbm.at[s * NGRP + j], ib_v.at[b],
                                     isems[b])

    _i(0, 0).start()
    _i(1, 1).start()
    plsc.subcore_barrier()

    def body(i, carry):
        g0 = 2 * i
        _i(g0, 0).wait()
        pltpu.sync_copy(obuf_v, acc_sh.at[ib_v.at[0].at[pl.ds(GC, GC)]],
                        add=True)
        _i(g0 + 2, 0).start()
        _i(g0 + 1, 1).wait()
        pltpu.sync_copy(obuf_v, acc_sh.at[ib_v.at[1].at[pl.ds(GC, GC)]],
                        add=True)
        _i(g0 + 3, 1).start()
        return carry

    lax.fori_loop(0, NGRP // 2 - 1, body, 0)
    for b in range(2):
        _i(NGRP - 2 + b, b).wait()
        pltpu.sync_copy(obuf_v, acc_sh.at[ib_v.at[b].at[pl.ds(GC, GC)]],
                        add=True)
    plsc.subcore_barrier()
    pltpu.sync_copy(acc_sh.at[pl.ds(s * SLAB, SLAB)],
                    out_hbm.at[c].at[pl.ds(s * SLAB, SLAB)])


_sc_deg = pl.kernel(
    _sc_deg_body,
    out_type=jax.ShapeDtypeStruct((2, NPAD, DH), jnp.float32),
    mesh=plsc.VectorSubcoreMesh(core_axis_name="c", subcore_axis_name="s"),
    compiler_params=pltpu.CompilerParams(use_tc_tiling_on_sc=False),
    scratch_types=[
        pltpu.VMEM((2, PK), jnp.int32),
        pltpu.VMEM((GC, DH), jnp.float32),
        pltpu.VMEM_SHARED((NPAD, DH), jnp.float32),
        pltpu.SemaphoreType.DMA,
        pltpu.SemaphoreType.DMA,
    ],
)


def _split_u(u_ref, v):
    u_ref[0] = v[:, :DH]
    u_ref[1] = v[:, DH:]


def _tc_prep_body(degp_ref, x_ref, dinv_ref, u0_ref):
    deg = degp_ref[0, :N, 0]
    dinv = jnp.where(deg > 0, lax.rsqrt(jnp.maximum(deg, 1e-12)), 0.0)
    dinv = dinv[:, None]
    dinv_ref[...] = dinv
    _split_u(u0_ref, dinv * x_ref[...])


def _tc_combine_a_u_body(p_ref, dinv_ref, tx_ref, u_ref):
    st = jnp.concatenate([p_ref[0, :N, :], p_ref[1, :N, :]], axis=1)
    dinv = dinv_ref[...]
    tx1 = -dinv * st
    tx_ref[...] = tx1
    _split_u(u_ref, dinv * tx1)


def _tc_combine_a_mm_body(h_ref, tx_ref, w0_ref, w1_ref, acc_ref):
    acc_ref[...] = (jnp.dot(h_ref[...], w0_ref[...],
                            preferred_element_type=jnp.float32)
                    + jnp.dot(tx_ref[...], w1_ref[...],
                              preferred_element_type=jnp.float32))


def _tc_combine_b_u_body(p_ref, dinv_ref, prev2_ref, tx_ref, u_ref):
    st = jnp.concatenate([p_ref[0, :N, :], p_ref[1, :N, :]], axis=1)
    dinv = dinv_ref[...]
    txk = -2.0 * dinv * st - prev2_ref[...]
    tx_ref[...] = txk
    _split_u(u_ref, dinv * txk)


def _tc_combine_b_mm_body(acc_in_ref, tx_ref, wk_ref, acc_ref):
    acc_ref[...] = acc_in_ref[...] + jnp.dot(
        tx_ref[...], wk_ref[...], preferred_element_type=jnp.float32)


def _tc_combine_b_tail_body(p_ref, dinv_ref, prev2_ref, wk_ref, acc_in_ref,
                            b_ref, g_ref, be_ref, h_ref, u_ref):
    st = jnp.concatenate([p_ref[0, :N, :], p_ref[1, :N, :]], axis=1)
    dinv = dinv_ref[...]
    txk = -2.0 * dinv * st - prev2_ref[...]
    acc = acc_in_ref[...] + jnp.dot(
        txk, wk_ref[...], preferred_element_type=jnp.float32)
    h = jnp.maximum(acc + b_ref[...][None, :], 0.0)
    m = jnp.mean(h, axis=0, keepdims=True)
    v = jnp.mean((h - m) * (h - m), axis=0, keepdims=True)
    h = (h - m) * lax.rsqrt(v + 1e-5) * g_ref[...][None, :] + be_ref[...][None, :]
    h_ref[...] = h
    _split_u(u_ref, dinv * h)


def _tc_combine_b_final_body(p_ref, dinv_ref, prev2_ref, wk_ref, acc_in_ref,
                             b4_ref, mw0_ref, mg_ref, mbe_ref, mw1_ref,
                             mb1_ref, out_ref):
    st = jnp.concatenate([p_ref[0, :N, :], p_ref[1, :N, :]], axis=1)
    dinv = dinv_ref[...]
    txk = -2.0 * dinv * st - prev2_ref[...]
    acc = acc_in_ref[...] + jnp.dot(
        txk, wk_ref[...], preferred_element_type=jnp.float32)
    h4 = acc + b4_ref[...][None, :]
    z = jnp.dot(h4, mw0_ref[...], preferred_element_type=jnp.float32)
    m = jnp.mean(z, axis=0, keepdims=True)
    v = jnp.mean((z - m) * (z - m), axis=0, keepdims=True)
    z = (z - m) * lax.rsqrt(v + 1e-5) * mg_ref[...][None, :] + mbe_ref[...][None, :]
    h2 = jnp.maximum(z, 0.0)
    out_ref[...] = (jnp.dot(h2, mw1_ref[...], preferred_element_type=jnp.float32)
                    + mb1_ref[...][None, :])


def _tc(body, out_shapes):
    return pl.pallas_call(body, out_shape=out_shapes)


_F = jnp.float32
_U2 = jax.ShapeDtypeStruct((2, N, DH), _F)
_prep = _tc(_tc_prep_body, (jax.ShapeDtypeStruct((N, 1), _F), _U2))
_NF = jax.ShapeDtypeStruct((N, DF), _F)
_combine_a_u = _tc(_tc_combine_a_u_body, (_NF, _U2))
_combine_a_mm = _tc(_tc_combine_a_mm_body, _NF)
_combine_b_u = _tc(_tc_combine_b_u_body, (_NF, _U2))
_combine_b_mm = _tc(_tc_combine_b_mm_body, _NF)
_combine_bt = _tc(_tc_combine_b_tail_body,
                  (jax.ShapeDtypeStruct((N, DF), _F), _U2))
_combine_bf = _tc(_tc_combine_b_final_body, jax.ShapeDtypeStruct((N, 21), _F))


def kernel(x, edge_index, batch, W1, b1, W2, b2, W3, b3, W4, b4,
           g1, be1, g2, be2, g3, be3, mw0, mg, mbe, mw1, mb1):
    del batch
    pad = EPAD - edge_index.shape[1]
    rowp = jnp.concatenate(
        [edge_index[0].astype(jnp.int32), jnp.zeros((pad,), jnp.int32)])
    colp = jnp.concatenate(
        [edge_index[1].astype(jnp.int32), jnp.full((pad,), N, jnp.int32)])
    pk = jnp.concatenate([rowp.reshape(16 * NGRP, 1, GC),
                          colp.reshape(16 * NGRP, 1, GC)],
                         axis=1).reshape(16 * NGRP, PK)
    zrows = jnp.zeros((SLAB, DH), _F)
    onesg = jnp.ones((GC, DH), _F)

    degp = _sc_deg(onesg, pk, zrows)
    dinv, u = _prep(degp, x)

    h = x
    Ws = (W1, W2, W3, W4)
    bs = (b1, b2, b3, b4)
    gs = (g1, g2, g3)
    bes = (be1, be2, be3)
    for l in range(4):
        W = Ws[l]
        p = _sc_prop(u, pk, zrows)
        tx1, u = _combine_a_u(p, dinv)
        acc = _combine_a_mm(h, tx1, W[0], W[1])
        p = _sc_prop(u, pk, zrows)
        tx2, u = _combine_b_u(p, dinv, h)
        acc = _combine_b_mm(acc, tx2, W[2])
        p = _sc_prop(u, pk, zrows)
        if l < 3:
            h, u = _combine_bt(p, dinv, tx1, W[3], acc,
                               bs[l], gs[l], bes[l])
        else:
            out = _combine_bf(p, dinv, tx1, W[3], acc,
                              bs[l], mw0, mg, mbe, mw1, mb1)
    return out
